# baseline (speedup 1.0000x reference)
import functools

import jax
import jax.numpy as jnp
from jax import lax
from jax.experimental import pallas as pl
from jax.experimental.pallas import tpu as pltpu
from jax.experimental.pallas import tpu_sc as plsc

BATCH = 16384
EMBED = 64
HID = 32

NUM_CORES = 2
NUM_SUBCORES = 16
NUM_WORKERS = NUM_CORES * NUM_SUBCORES
B_PER_W = BATCH // NUM_WORKERS

ROW_BLOCK = 2048


def _gather_body(uid_hbm, iid_hbm, utab_hbm, itab_hbm, uout_hbm, iout_hbm,
                 uidx_v, urows_v, iidx_v, irows_v, usem, isem):
    wid = lax.axis_index("s") * NUM_CORES + lax.axis_index("c")
    base = wid * B_PER_W
    pltpu.sync_copy(uid_hbm.at[pl.ds(base, B_PER_W)], uidx_v)
    pltpu.sync_copy(iid_hbm.at[pl.ds(base, B_PER_W)], iidx_v)
    ucp = pltpu.async_copy(utab_hbm.at[uidx_v], urows_v, usem)
    icp = pltpu.async_copy(itab_hbm.at[iidx_v], irows_v, isem)
    ucp.wait()
    icp.wait()
    pltpu.sync_copy(urows_v, uout_hbm.at[pl.ds(base, B_PER_W)])
    pltpu.sync_copy(irows_v, iout_hbm.at[pl.ds(base, B_PER_W)])


_sc_gather = functools.partial(
    pl.kernel,
    out_type=[
        jax.ShapeDtypeStruct((BATCH, EMBED), jnp.float32),
        jax.ShapeDtypeStruct((BATCH, EMBED), jnp.float32),
    ],
    mesh=plsc.VectorSubcoreMesh(core_axis_name="c", subcore_axis_name="s"),
    compiler_params=pltpu.CompilerParams(use_tc_tiling_on_sc=False),
    scratch_types=[
        pltpu.VMEM((B_PER_W,), jnp.int32),
        pltpu.VMEM((B_PER_W, EMBED), jnp.float32),
        pltpu.VMEM((B_PER_W,), jnp.int32),
        pltpu.VMEM((B_PER_W, EMBED), jnp.float32),
        pltpu.SemaphoreType.DMA,
        pltpu.SemaphoreType.DMA,
    ],
)(_gather_body)


def _mlp_body(u_emb_ref, i_emb_ref, uW1_ref, ub1_ref, uW2_ref, ub2_ref,
              iW1_ref, ib1_ref, iW2_ref, ib2_ref, out_ref):
    ue = u_emb_ref[...]
    uh = jnp.maximum(
        jnp.dot(ue, uW1_ref[...], preferred_element_type=jnp.float32)
        + ub1_ref[...], 0.0)
    uv = jnp.dot(uh, uW2_ref[...], preferred_element_type=jnp.float32) \
        + ub2_ref[...]
    ie = i_emb_ref[...]
    ih = jnp.maximum(
        jnp.dot(ie, iW1_ref[...], preferred_element_type=jnp.float32)
        + ib1_ref[...], 0.0)
    iv = jnp.dot(ih, iW2_ref[...], preferred_element_type=jnp.float32) \
        + ib2_ref[...]
    un = jnp.sqrt(jnp.sum(uv * uv, axis=1))
    iN = jnp.sqrt(jnp.sum(iv * iv, axis=1))
    dot = jnp.sum(uv * iv, axis=1)
    eps = jnp.float32(1e-12)
    out_ref[...] = dot / (jnp.maximum(un, eps) * jnp.maximum(iN, eps))


def _mlp_call(u_emb, i_emb, uW1, ub1, uW2, ub2, iW1, ib1, iW2, ib2):
    n_blocks = BATCH // ROW_BLOCK
    w_spec = lambda shape: pl.BlockSpec(shape, lambda i: (0,) * len(shape))
    return pl.pallas_call(
        _mlp_body,
        grid=(n_blocks,),
        in_specs=[
            pl.BlockSpec((ROW_BLOCK, EMBED), lambda i: (i, 0)),
            pl.BlockSpec((ROW_BLOCK, EMBED), lambda i: (i, 0)),
            w_spec((EMBED, HID)),
            w_spec((1, HID)),
            w_spec((HID, HID)),
            w_spec((1, HID)),
            w_spec((EMBED, HID)),
            w_spec((1, HID)),
            w_spec((HID, HID)),
            w_spec((1, HID)),
        ],
        out_specs=pl.BlockSpec((ROW_BLOCK,), lambda i: (i,)),
        out_shape=jax.ShapeDtypeStruct((BATCH,), jnp.float32),
    )(u_emb, i_emb, uW1, ub1, uW2, ub2, iW1, ib1, iW2, ib2)


def kernel(user_ids, item_ids, user_table, item_table,
           uW1, ub1, uW2, ub2, iW1, ib1, iW2, ib2):
    u_emb, i_emb = _sc_gather(user_ids, item_ids, user_table, item_table)
    return _mlp_call(u_emb, i_emb,
                     uW1, ub1.reshape(1, HID), uW2, ub2.reshape(1, HID),
                     iW1, ib1.reshape(1, HID), iW2, ib2.reshape(1, HID))


# speedup vs baseline: 1.0014x; 1.0014x over previous
import functools

import jax
import jax.numpy as jnp
from jax import lax
from jax.experimental import pallas as pl
from jax.experimental.pallas import tpu as pltpu
from jax.experimental.pallas import tpu_sc as plsc

BATCH = 16384
EMBED = 64
HID = 32
NPAIR = 500000
PAIR_W = 2 * EMBED

NUM_CORES = 2
NUM_SUBCORES = 16
NUM_WORKERS = NUM_CORES * NUM_SUBCORES
B_PER_W = BATCH // NUM_WORKERS

CHUNK = 128
NCHUNK = B_PER_W // CHUNK

ROW_BLOCK = 2048


def _gather_body(upair_hbm, ipair_hbm, utab_hbm, itab_hbm, uout_hbm, iout_hbm,
                 uidx_v, iidx_v, ubuf0, ubuf1, ibuf0, ibuf1,
                 ugsem, igsem, uwsem, iwsem):
    wid = lax.axis_index("s") * NUM_CORES + lax.axis_index("c")
    base = wid * B_PER_W
    pltpu.sync_copy(upair_hbm.at[pl.ds(base, B_PER_W)], uidx_v)
    pltpu.sync_copy(ipair_hbm.at[pl.ds(base, B_PER_W)], iidx_v)

    ubufs = (ubuf0, ubuf1)
    ibufs = (ibuf0, ibuf1)

    def gather(c, b):
        ucp = pltpu.async_copy(
            utab_hbm.at[uidx_v.at[pl.ds(c * CHUNK, CHUNK)]], ubufs[b], ugsem)
        icp = pltpu.async_copy(
            itab_hbm.at[iidx_v.at[pl.ds(c * CHUNK, CHUNK)]], ibufs[b], igsem)
        return ucp, icp

    def write(c, b):
        dst = pl.ds(base + c * CHUNK, CHUNK)
        ucp = pltpu.async_copy(ubufs[b], uout_hbm.at[dst], uwsem)
        icp = pltpu.async_copy(ibufs[b], iout_hbm.at[dst], iwsem)
        return ucp, icp

    pend_g = gather(0, 0)
    pend_w = [None, None]
    for c in range(NCHUNK):
        b = c & 1
        pend_g[0].wait()
        pend_g[1].wait()
        if c + 1 < NCHUNK:
            if pend_w[b ^ 1] is not None:
                pend_w[b ^ 1][0].wait()
                pend_w[b ^ 1][1].wait()
            pend_g = gather(c + 1, b ^ 1)
        pend_w[b] = write(c, b)
    for pw in pend_w:
        if pw is not None:
            pw[0].wait()
            pw[1].wait()


_sc_gather = functools.partial(
    pl.kernel,
    out_type=[
        jax.ShapeDtypeStruct((BATCH, PAIR_W), jnp.float32),
        jax.ShapeDtypeStruct((BATCH, PAIR_W), jnp.float32),
    ],
    mesh=plsc.VectorSubcoreMesh(core_axis_name="c", subcore_axis_name="s"),
    scratch_types=[
        pltpu.VMEM((B_PER_W,), jnp.int32),
        pltpu.VMEM((B_PER_W,), jnp.int32),
        pltpu.VMEM((CHUNK, PAIR_W), jnp.float32),
        pltpu.VMEM((CHUNK, PAIR_W), jnp.float32),
        pltpu.VMEM((CHUNK, PAIR_W), jnp.float32),
        pltpu.VMEM((CHUNK, PAIR_W), jnp.float32),
        pltpu.SemaphoreType.DMA,
        pltpu.SemaphoreType.DMA,
        pltpu.SemaphoreType.DMA,
        pltpu.SemaphoreType.DMA,
    ],
)(_gather_body)


def _mlp_body(u2_ref, i2_ref, hu_ref, hi_ref, uW1_ref, ub1_ref, uW2_ref,
              ub2_ref, iW1_ref, ib1_ref, iW2_ref, ib2_ref, out_ref):
    hu = hu_ref[...]
    hi = hi_ref[...]
    ue = u2_ref[:, :EMBED] * (1.0 - hu) + u2_ref[:, EMBED:] * hu
    ie = i2_ref[:, :EMBED] * (1.0 - hi) + i2_ref[:, EMBED:] * hi
    uh = jnp.maximum(
        jnp.dot(ue, uW1_ref[...], preferred_element_type=jnp.float32)
        + ub1_ref[...], 0.0)
    uv = jnp.dot(uh, uW2_ref[...], preferred_element_type=jnp.float32) \
        + ub2_ref[...]
    ih = jnp.maximum(
        jnp.dot(ie, iW1_ref[...], preferred_element_type=jnp.float32)
        + ib1_ref[...], 0.0)
    iv = jnp.dot(ih, iW2_ref[...], preferred_element_type=jnp.float32) \
        + ib2_ref[...]
    un = jnp.sqrt(jnp.sum(uv * uv, axis=1))
    iN = jnp.sqrt(jnp.sum(iv * iv, axis=1))
    dot = jnp.sum(uv * iv, axis=1)
    eps = jnp.float32(1e-12)
    out_ref[...] = dot / (jnp.maximum(un, eps) * jnp.maximum(iN, eps))


def _mlp_call(u2, i2, hu, hi, uW1, ub1, uW2, ub2, iW1, ib1, iW2, ib2):
    n_blocks = BATCH // ROW_BLOCK
    w_spec = lambda shape: pl.BlockSpec(shape, lambda i: (0,) * len(shape))
    return pl.pallas_call(
        _mlp_body,
        grid=(n_blocks,),
        in_specs=[
            pl.BlockSpec((ROW_BLOCK, PAIR_W), lambda i: (i, 0)),
            pl.BlockSpec((ROW_BLOCK, PAIR_W), lambda i: (i, 0)),
            pl.BlockSpec((ROW_BLOCK, 1), lambda i: (i, 0)),
            pl.BlockSpec((ROW_BLOCK, 1), lambda i: (i, 0)),
            w_spec((EMBED, HID)),
            w_spec((1, HID)),
            w_spec((HID, HID)),
            w_spec((1, HID)),
            w_spec((EMBED, HID)),
            w_spec((1, HID)),
            w_spec((HID, HID)),
            w_spec((1, HID)),
        ],
        out_specs=pl.BlockSpec((ROW_BLOCK,), lambda i: (i,)),
        out_shape=jax.ShapeDtypeStruct((BATCH,), jnp.float32),
    )(u2, i2, hu, hi, uW1, ub1, uW2, ub2, iW1, ib1, iW2, ib2)


def kernel(user_ids, item_ids, user_table, item_table,
           uW1, ub1, uW2, ub2, iW1, ib1, iW2, ib2):
    utab2 = user_table.reshape(NPAIR, PAIR_W)
    itab2 = item_table.reshape(NPAIR, PAIR_W)
    upair = lax.shift_right_logical(user_ids, 1)
    ipair = lax.shift_right_logical(item_ids, 1)
    hu = (user_ids & 1).astype(jnp.float32).reshape(BATCH, 1)
    hi = (item_ids & 1).astype(jnp.float32).reshape(BATCH, 1)
    u2, i2 = _sc_gather(upair, ipair, utab2, itab2)
    return _mlp_call(u2, i2, hu, hi,
                     uW1, ub1.reshape(1, HID), uW2, ub2.reshape(1, HID),
                     iW1, ib1.reshape(1, HID), iW2, ib2.reshape(1, HID))
